# Initial kernel scaffold; baseline (speedup 1.0000x reference)
import jax
import jax.numpy as jnp
from jax.experimental import pallas as pl


def kernel(inputs, edge_index, W1, b1, W2, b2):
    raise NotImplementedError("write your pallas kernel here")


# speedup vs baseline: 8.2113x; 8.2113x over previous
import functools

import jax
import jax.numpy as jnp
from jax import lax
from jax.experimental import pallas as pl
from jax.experimental.pallas import tpu as pltpu
from jax.experimental.pallas import tpu_sc as plsc

N = 50000
E = 800000
D = 64
H = 32

NP = 50176
ROWS_PER_TILE = NP // 16
EP = 802816
CHUNKS = 1568
CHUNKS_PER_TILE = CHUNKS // 16
DCHUNKS = 784
DCHUNKS_PER_TILE = DCHUNKS // 16
PAD_BASE = 50048

_mesh = plsc.VectorSubcoreMesh(core_axis_name="c", subcore_axis_name="s")
_sc_params = pltpu.CompilerParams(use_tc_tiling_on_sc=False)


def _zero_rows_vmem(rows_v, n_rows):
    z = jnp.zeros((16,), jnp.float32)

    def body(i, _):
        rows_v[i, pl.ds(0, 16)] = z
        rows_v[i, pl.ds(16, 16)] = z
        return 0

    lax.fori_loop(0, n_rows, body, 0)


@functools.partial(
    pl.kernel,
    out_type=(
        jax.ShapeDtypeStruct((NP,), jnp.float32),
        jax.ShapeDtypeStruct((NP,), jnp.float32),
    ),
    mesh=_mesh,
    scratch_types=(
        pltpu.VMEM((8, 128), jnp.int32),
        pltpu.VMEM((128,), jnp.float32),
        pltpu.VMEM((ROWS_PER_TILE,), jnp.float32),
        pltpu.VMEM_SHARED((NP,), jnp.float32),
    ),
    compiler_params=_sc_params,
)
def _degree_kernel(src_hbm, dst_hbm, dego_hbm, degi_hbm, idx_v, ones_v, buf_v, acc_sh):
    cid = lax.axis_index("c")
    sid = lax.axis_index("s")

    one = jnp.ones((16,), jnp.float32)
    zero = jnp.zeros((16,), jnp.float32)
    for j in range(8):
        ones_v[pl.ds(16 * j, 16)] = one

    def zbody(i, _):
        buf_v[pl.ds(16 * i, 16)] = zero
        return 0

    lax.fori_loop(0, ROWS_PER_TILE // 16, zbody, 0)

    base = sid * ROWS_PER_TILE
    pltpu.sync_copy(buf_v, acc_sh.at[pl.ds(base, ROWS_PER_TILE)])
    plsc.subcore_barrier()

    def accumulate(edge_hbm):
        def body(c, _):
            g = sid * DCHUNKS_PER_TILE + c
            pltpu.sync_copy(edge_hbm.at[g], idx_v)
            for j in range(8):
                pltpu.sync_copy(ones_v, acc_sh.at[idx_v.at[j]], add=True)
            return 0

        lax.fori_loop(0, DCHUNKS_PER_TILE, body, 0)

    @pl.when(cid == 0)
    def _():
        accumulate(src_hbm)

    @pl.when(cid == 1)
    def _():
        accumulate(dst_hbm)

    plsc.subcore_barrier()

    pltpu.sync_copy(acc_sh.at[pl.ds(base, ROWS_PER_TILE)], buf_v)

    @pl.when(cid == 0)
    def _():
        pltpu.sync_copy(buf_v, dego_hbm.at[pl.ds(base, ROWS_PER_TILE)])

    @pl.when(cid == 1)
    def _():
        pltpu.sync_copy(buf_v, degi_hbm.at[pl.ds(base, ROWS_PER_TILE)])


@functools.partial(
    pl.kernel,
    out_type=(
        jax.ShapeDtypeStruct((NP, H), jnp.float32),
        jax.ShapeDtypeStruct((NP, H), jnp.float32),
    ),
    mesh=_mesh,
    scratch_types=(
        pltpu.VMEM((4, 128), jnp.int32),
        pltpu.VMEM((4, 128), jnp.int32),
        pltpu.VMEM((512, H), jnp.float32),
        pltpu.VMEM_SHARED((NP, H), jnp.float32),
        pltpu.SemaphoreType.DMA,
    ),
    compiler_params=_sc_params,
)
def _message_kernel(h0_hbm, h1_hbm, src_hbm, dst_hbm, agg0_hbm, agg1_hbm,
                    src_v, dst_v, rows_v, acc_sh, sem):
    cid = lax.axis_index("c")
    sid = lax.axis_index("s")

    _zero_rows_vmem(rows_v, 512)
    base = sid * ROWS_PER_TILE
    for k in range(6):
        pltpu.sync_copy(rows_v, acc_sh.at[pl.ds(base + 512 * k, 512)])
    pltpu.sync_copy(rows_v.at[pl.ds(0, 64)],
                    acc_sh.at[pl.ds(base + 3072, 64)])
    plsc.subcore_barrier()

    def run(h_hbm):
        def body(c, _):
            g = sid * CHUNKS_PER_TILE + c
            pltpu.sync_copy(src_hbm.at[g], src_v)
            pltpu.sync_copy(dst_hbm.at[g], dst_v)
            descs = [
                pltpu.async_copy(h_hbm.at[src_v.at[j]],
                                 rows_v.at[pl.ds(128 * j, 128)], sem)
                for j in range(4)
            ]
            for d in descs:
                d.wait()
            for j in range(4):
                pltpu.sync_copy(rows_v.at[pl.ds(128 * j, 128)],
                                acc_sh.at[dst_v.at[j]], add=True)
            return 0

        lax.fori_loop(0, CHUNKS_PER_TILE, body, 0)

    @pl.when(cid == 0)
    def _():
        run(h0_hbm)

    @pl.when(cid == 1)
    def _():
        run(h1_hbm)

    plsc.subcore_barrier()

    def writeout(agg_hbm):
        for k in range(6):
            pltpu.sync_copy(acc_sh.at[pl.ds(base + 512 * k, 512)], rows_v)
            pltpu.sync_copy(rows_v, agg_hbm.at[pl.ds(base + 512 * k, 512)])
        pltpu.sync_copy(acc_sh.at[pl.ds(base + 3072, 64)],
                        rows_v.at[pl.ds(0, 64)])
        pltpu.sync_copy(rows_v.at[pl.ds(0, 64)],
                        agg_hbm.at[pl.ds(base + 3072, 64)])

    @pl.when(cid == 0)
    def _():
        writeout(agg0_hbm)

    @pl.when(cid == 1)
    def _():
        writeout(agg1_hbm)


_BLK = 2000
_GRID = N // _BLK


def _scale_body(x_ref, deg_ref, h0_ref, h1_ref):
    norm = lax.rsqrt(jnp.maximum(deg_ref[...], 1.0))
    h = x_ref[...] * norm
    h0_ref[...] = h[:, :H]
    h1_ref[...] = h[:, H:]


def _tc_scale(x, deg_out):
    return pl.pallas_call(
        _scale_body,
        grid=(_GRID,),
        in_specs=[
            pl.BlockSpec((_BLK, D), lambda i: (i, 0)),
            pl.BlockSpec((_BLK, 1), lambda i: (i, 0)),
        ],
        out_specs=[
            pl.BlockSpec((_BLK, H), lambda i: (i, 0)),
            pl.BlockSpec((_BLK, H), lambda i: (i, 0)),
        ],
        out_shape=[
            jax.ShapeDtypeStruct((N, H), jnp.float32),
            jax.ShapeDtypeStruct((N, H), jnp.float32),
        ],
    )(x, deg_out)


def _mid_body(a0_ref, a1_ref, din_ref, dout_ref, w_ref, b_ref, h0_ref, h1_ref):
    nd = lax.rsqrt(jnp.maximum(din_ref[...], 1.0))
    ns = lax.rsqrt(jnp.maximum(dout_ref[...], 1.0))
    a0 = a0_ref[...] * nd
    a1 = a1_ref[...] * nd
    h = (jnp.dot(a0, w_ref[:H, :], preferred_element_type=jnp.float32)
         + jnp.dot(a1, w_ref[H:, :], preferred_element_type=jnp.float32)
         + b_ref[...])
    h = jnp.maximum(h, 0.0) * ns
    h0_ref[...] = h[:, :H]
    h1_ref[...] = h[:, H:]


def _tc_mid(agg0, agg1, deg_in, deg_out, w, b):
    return pl.pallas_call(
        _mid_body,
        grid=(_GRID,),
        in_specs=[
            pl.BlockSpec((_BLK, H), lambda i: (i, 0)),
            pl.BlockSpec((_BLK, H), lambda i: (i, 0)),
            pl.BlockSpec((_BLK, 1), lambda i: (i, 0)),
            pl.BlockSpec((_BLK, 1), lambda i: (i, 0)),
            pl.BlockSpec((D, D), lambda i: (0, 0)),
            pl.BlockSpec((1, D), lambda i: (0, 0)),
        ],
        out_specs=[
            pl.BlockSpec((_BLK, H), lambda i: (i, 0)),
            pl.BlockSpec((_BLK, H), lambda i: (i, 0)),
        ],
        out_shape=[
            jax.ShapeDtypeStruct((N, H), jnp.float32),
            jax.ShapeDtypeStruct((N, H), jnp.float32),
        ],
    )(agg0, agg1, deg_in, deg_out, w, b)


def _final_body(a0_ref, a1_ref, din_ref, w_ref, b_ref, out_ref):
    nd = lax.rsqrt(jnp.maximum(din_ref[...], 1.0))
    a0 = a0_ref[...] * nd
    a1 = a1_ref[...] * nd
    out_ref[...] = (jnp.dot(a0, w_ref[:H, :], preferred_element_type=jnp.float32)
                    + jnp.dot(a1, w_ref[H:, :], preferred_element_type=jnp.float32)
                    + b_ref[...])


def _tc_final(agg0, agg1, deg_in, w, b):
    return pl.pallas_call(
        _final_body,
        grid=(_GRID,),
        in_specs=[
            pl.BlockSpec((_BLK, H), lambda i: (i, 0)),
            pl.BlockSpec((_BLK, H), lambda i: (i, 0)),
            pl.BlockSpec((_BLK, 1), lambda i: (i, 0)),
            pl.BlockSpec((D, D), lambda i: (0, 0)),
            pl.BlockSpec((1, D), lambda i: (0, 0)),
        ],
        out_specs=pl.BlockSpec((_BLK, D), lambda i: (i, 0)),
        out_shape=jax.ShapeDtypeStruct((N, D), jnp.float32),
    )(agg0, agg1, deg_in, w, b)


def kernel(inputs, edge_index, W1, b1, W2, b2):
    src = edge_index[0]
    dst = edge_index[1]
    padn = EP - E
    pad_ids = PAD_BASE + (jnp.arange(padn, dtype=jnp.int32) % 128)
    zeros_pad = jnp.zeros((padn,), jnp.int32)
    src_g = jnp.concatenate([src, zeros_pad]).reshape(CHUNKS, 4, 128)
    src_d = jnp.concatenate([src, pad_ids]).reshape(DCHUNKS, 8, 128)
    dst_sd = jnp.concatenate([dst, pad_ids]).reshape(DCHUNKS, 8, 128)
    dst_s = jnp.concatenate([dst, pad_ids]).reshape(CHUNKS, 4, 128)

    deg_out, deg_in = _degree_kernel(src_d, dst_sd)
    deg_out = deg_out.reshape(NP, 1)
    deg_in = deg_in.reshape(NP, 1)

    b1r = b1.reshape(1, D)
    b2r = b2.reshape(1, D)

    h0, h1 = _tc_scale(inputs, deg_out)
    agg0, agg1 = _message_kernel(h0, h1, src_g, dst_s)
    h20, h21 = _tc_mid(agg0, agg1, deg_in, deg_out, W1, b1r)
    agg20, agg21 = _message_kernel(h20, h21, src_g, dst_s)
    return _tc_final(agg20, agg21, deg_in, W2, b2r)


# speedup vs baseline: 8.5657x; 1.0432x over previous
import functools

import jax
import jax.numpy as jnp
from jax import lax
from jax.experimental import pallas as pl
from jax.experimental.pallas import tpu as pltpu
from jax.experimental.pallas import tpu_sc as plsc

N = 50000
E = 800000
D = 64
H = 32

NP = 50176
ROWS_PER_TILE = NP // 16
EP = 802816
CHUNKS = 3136
CHUNKS_PER_TILE = CHUNKS // 16
DCHUNKS = 784
DCHUNKS_PER_TILE = DCHUNKS // 16
PAD_BASE = 50048

_mesh = plsc.VectorSubcoreMesh(core_axis_name="c", subcore_axis_name="s")
_sc_params = pltpu.CompilerParams(use_tc_tiling_on_sc=False)


def _zero_rows_vmem(rows_v, n_rows):
    z = jnp.zeros((16,), jnp.float32)

    def body(i, _):
        rows_v[i, pl.ds(0, 16)] = z
        rows_v[i, pl.ds(16, 16)] = z
        return 0

    lax.fori_loop(0, n_rows, body, 0)


@functools.partial(
    pl.kernel,
    out_type=(
        jax.ShapeDtypeStruct((NP,), jnp.float32),
        jax.ShapeDtypeStruct((NP,), jnp.float32),
    ),
    mesh=_mesh,
    scratch_types=(
        pltpu.VMEM((8, 128), jnp.int32),
        pltpu.VMEM((128,), jnp.float32),
        pltpu.VMEM((ROWS_PER_TILE,), jnp.float32),
        pltpu.VMEM_SHARED((NP,), jnp.float32),
    ),
    compiler_params=_sc_params,
)
def _degree_kernel(src_hbm, dst_hbm, dego_hbm, degi_hbm, idx_v, ones_v, buf_v, acc_sh):
    cid = lax.axis_index("c")
    sid = lax.axis_index("s")

    one = jnp.ones((16,), jnp.float32)
    zero = jnp.zeros((16,), jnp.float32)
    for j in range(8):
        ones_v[pl.ds(16 * j, 16)] = one

    def zbody(i, _):
        buf_v[pl.ds(16 * i, 16)] = zero
        return 0

    lax.fori_loop(0, ROWS_PER_TILE // 16, zbody, 0)

    base = sid * ROWS_PER_TILE
    pltpu.sync_copy(buf_v, acc_sh.at[pl.ds(base, ROWS_PER_TILE)])
    plsc.subcore_barrier()

    def accumulate(edge_hbm):
        def body(c, _):
            g = sid * DCHUNKS_PER_TILE + c
            pltpu.sync_copy(edge_hbm.at[g], idx_v)
            for j in range(8):
                pltpu.sync_copy(ones_v, acc_sh.at[idx_v.at[j]], add=True)
            return 0

        lax.fori_loop(0, DCHUNKS_PER_TILE, body, 0)

    @pl.when(cid == 0)
    def _():
        accumulate(src_hbm)

    @pl.when(cid == 1)
    def _():
        accumulate(dst_hbm)

    plsc.subcore_barrier()

    pltpu.sync_copy(acc_sh.at[pl.ds(base, ROWS_PER_TILE)], buf_v)

    @pl.when(cid == 0)
    def _():
        pltpu.sync_copy(buf_v, dego_hbm.at[pl.ds(base, ROWS_PER_TILE)])

    @pl.when(cid == 1)
    def _():
        pltpu.sync_copy(buf_v, degi_hbm.at[pl.ds(base, ROWS_PER_TILE)])


@functools.partial(
    pl.kernel,
    out_type=(
        jax.ShapeDtypeStruct((NP, H), jnp.float32),
        jax.ShapeDtypeStruct((NP, H), jnp.float32),
    ),
    mesh=_mesh,
    scratch_types=(
        pltpu.VMEM((2, 128), jnp.int32),
        pltpu.VMEM((2, 128), jnp.int32),
        pltpu.VMEM((2, 128), jnp.int32),
        pltpu.VMEM((2, 128), jnp.int32),
        pltpu.VMEM((256, H), jnp.float32),
        pltpu.VMEM((256, H), jnp.float32),
        pltpu.VMEM_SHARED((NP, H), jnp.float32),
        pltpu.SemaphoreType.DMA,
        pltpu.SemaphoreType.DMA,
        pltpu.SemaphoreType.DMA,
        pltpu.SemaphoreType.DMA,
    ),
    compiler_params=_sc_params,
)
def _message_kernel(h0_hbm, h1_hbm, src_hbm, dst_hbm, agg0_hbm, agg1_hbm,
                    srcA, dstA, srcB, dstB, rowsA, rowsB, acc_sh,
                    gsA, gsB, ssA, ssB):
    cid = lax.axis_index("c")
    sid = lax.axis_index("s")

    _zero_rows_vmem(rowsA, 256)
    base = sid * ROWS_PER_TILE
    for k in range(12):
        pltpu.sync_copy(rowsA, acc_sh.at[pl.ds(base + 256 * k, 256)])
    pltpu.sync_copy(rowsA.at[pl.ds(0, 64)],
                    acc_sh.at[pl.ds(base + 3072, 64)])
    plsc.subcore_barrier()

    def run(h_hbm):
        def gather_descs(src_v, rows_v, gsem):
            return [pltpu.make_async_copy(h_hbm.at[src_v.at[j]],
                                          rows_v.at[pl.ds(128 * j, 128)], gsem)
                    for j in range(2)]

        def scatter_start(rows_v, dst_v, ssem):
            for j in range(2):
                pltpu.async_copy(rows_v.at[pl.ds(128 * j, 128)],
                                 acc_sh.at[dst_v.at[j]], ssem, add=True)

        def scatter_wait(rows_v, dst_v, ssem):
            for j in range(2):
                pltpu.make_async_copy(rows_v.at[pl.ds(128 * j, 128)],
                                      acc_sh.at[dst_v.at[j]], ssem).wait()

        def stage_and_gather(c, src_v, dst_v, rows_v, gsem):
            g = sid * CHUNKS_PER_TILE + c
            pltpu.sync_copy(src_hbm.at[g], src_v)
            pltpu.sync_copy(dst_hbm.at[g], dst_v)
            for d in gather_descs(src_v, rows_v, gsem):
                d.start()

        stage_and_gather(0, srcA, dstA, rowsA, gsA)

        def body(k, _):
            @pl.when(k > 0)
            def _():
                scatter_wait(rowsB, dstB, ssB)
            stage_and_gather(2 * k + 1, srcB, dstB, rowsB, gsB)
            for d in gather_descs(srcA, rowsA, gsA):
                d.wait()
            scatter_start(rowsA, dstA, ssA)
            for d in gather_descs(srcB, rowsB, gsB):
                d.wait()
            scatter_start(rowsB, dstB, ssB)
            scatter_wait(rowsA, dstA, ssA)
            nxt = lax.rem(2 * k + 2, CHUNKS_PER_TILE)
            stage_and_gather(nxt, srcA, dstA, rowsA, gsA)
            return 0

        lax.fori_loop(0, CHUNKS_PER_TILE // 2, body, 0)
        scatter_wait(rowsB, dstB, ssB)
        for d in gather_descs(srcA, rowsA, gsA):
            d.wait()

    @pl.when(cid == 0)
    def _():
        run(h0_hbm)

    @pl.when(cid == 1)
    def _():
        run(h1_hbm)

    plsc.subcore_barrier()

    def writeout(agg_hbm):
        for k in range(12):
            pltpu.sync_copy(acc_sh.at[pl.ds(base + 256 * k, 256)], rowsA)
            pltpu.sync_copy(rowsA, agg_hbm.at[pl.ds(base + 256 * k, 256)])
        pltpu.sync_copy(acc_sh.at[pl.ds(base + 3072, 64)],
                        rowsA.at[pl.ds(0, 64)])
        pltpu.sync_copy(rowsA.at[pl.ds(0, 64)],
                        agg_hbm.at[pl.ds(base + 3072, 64)])

    @pl.when(cid == 0)
    def _():
        writeout(agg0_hbm)

    @pl.when(cid == 1)
    def _():
        writeout(agg1_hbm)


_BLK = 2000
_GRID = N // _BLK


def _scale_body(x_ref, deg_ref, h0_ref, h1_ref):
    norm = lax.rsqrt(jnp.maximum(deg_ref[...], 1.0))
    h = x_ref[...] * norm
    h0_ref[...] = h[:, :H]
    h1_ref[...] = h[:, H:]


def _tc_scale(x, deg_out):
    return pl.pallas_call(
        _scale_body,
        grid=(_GRID,),
        in_specs=[
            pl.BlockSpec((_BLK, D), lambda i: (i, 0)),
            pl.BlockSpec((_BLK, 1), lambda i: (i, 0)),
        ],
        out_specs=[
            pl.BlockSpec((_BLK, H), lambda i: (i, 0)),
            pl.BlockSpec((_BLK, H), lambda i: (i, 0)),
        ],
        out_shape=[
            jax.ShapeDtypeStruct((N, H), jnp.float32),
            jax.ShapeDtypeStruct((N, H), jnp.float32),
        ],
    )(x, deg_out)


def _mid_body(a0_ref, a1_ref, din_ref, dout_ref, w_ref, b_ref, h0_ref, h1_ref):
    nd = lax.rsqrt(jnp.maximum(din_ref[...], 1.0))
    ns = lax.rsqrt(jnp.maximum(dout_ref[...], 1.0))
    a0 = a0_ref[...] * nd
    a1 = a1_ref[...] * nd
    h = (jnp.dot(a0, w_ref[:H, :], preferred_element_type=jnp.float32)
         + jnp.dot(a1, w_ref[H:, :], preferred_element_type=jnp.float32)
         + b_ref[...])
    h = jnp.maximum(h, 0.0) * ns
    h0_ref[...] = h[:, :H]
    h1_ref[...] = h[:, H:]


def _tc_mid(agg0, agg1, deg_in, deg_out, w, b):
    return pl.pallas_call(
        _mid_body,
        grid=(_GRID,),
        in_specs=[
            pl.BlockSpec((_BLK, H), lambda i: (i, 0)),
            pl.BlockSpec((_BLK, H), lambda i: (i, 0)),
            pl.BlockSpec((_BLK, 1), lambda i: (i, 0)),
            pl.BlockSpec((_BLK, 1), lambda i: (i, 0)),
            pl.BlockSpec((D, D), lambda i: (0, 0)),
            pl.BlockSpec((1, D), lambda i: (0, 0)),
        ],
        out_specs=[
            pl.BlockSpec((_BLK, H), lambda i: (i, 0)),
            pl.BlockSpec((_BLK, H), lambda i: (i, 0)),
        ],
        out_shape=[
            jax.ShapeDtypeStruct((N, H), jnp.float32),
            jax.ShapeDtypeStruct((N, H), jnp.float32),
        ],
    )(agg0, agg1, deg_in, deg_out, w, b)


def _final_body(a0_ref, a1_ref, din_ref, w_ref, b_ref, out_ref):
    nd = lax.rsqrt(jnp.maximum(din_ref[...], 1.0))
    a0 = a0_ref[...] * nd
    a1 = a1_ref[...] * nd
    out_ref[...] = (jnp.dot(a0, w_ref[:H, :], preferred_element_type=jnp.float32)
                    + jnp.dot(a1, w_ref[H:, :], preferred_element_type=jnp.float32)
                    + b_ref[...])


def _tc_final(agg0, agg1, deg_in, w, b):
    return pl.pallas_call(
        _final_body,
        grid=(_GRID,),
        in_specs=[
            pl.BlockSpec((_BLK, H), lambda i: (i, 0)),
            pl.BlockSpec((_BLK, H), lambda i: (i, 0)),
            pl.BlockSpec((_BLK, 1), lambda i: (i, 0)),
            pl.BlockSpec((D, D), lambda i: (0, 0)),
            pl.BlockSpec((1, D), lambda i: (0, 0)),
        ],
        out_specs=pl.BlockSpec((_BLK, D), lambda i: (i, 0)),
        out_shape=jax.ShapeDtypeStruct((N, D), jnp.float32),
    )(agg0, agg1, deg_in, w, b)


def kernel(inputs, edge_index, W1, b1, W2, b2):
    src = edge_index[0]
    dst = edge_index[1]
    padn = EP - E
    pad_ids = PAD_BASE + (jnp.arange(padn, dtype=jnp.int32) % 128)
    zeros_pad = jnp.zeros((padn,), jnp.int32)
    src_g = jnp.concatenate([src, zeros_pad]).reshape(CHUNKS, 2, 128)
    src_d = jnp.concatenate([src, pad_ids]).reshape(DCHUNKS, 8, 128)
    dst_sd = jnp.concatenate([dst, pad_ids]).reshape(DCHUNKS, 8, 128)
    dst_s = jnp.concatenate([dst, pad_ids]).reshape(CHUNKS, 2, 128)

    deg_out, deg_in = _degree_kernel(src_d, dst_sd)
    deg_out = deg_out.reshape(NP, 1)
    deg_in = deg_in.reshape(NP, 1)

    b1r = b1.reshape(1, D)
    b2r = b2.reshape(1, D)

    h0, h1 = _tc_scale(inputs, deg_out)
    agg0, agg1 = _message_kernel(h0, h1, src_g, dst_s)
    h20, h21 = _tc_mid(agg0, agg1, deg_in, deg_out, W1, b1r)
    agg20, agg21 = _message_kernel(h20, h21, src_g, dst_s)
    return _tc_final(agg20, agg21, deg_in, W2, b2r)


# speedup vs baseline: 11.0394x; 1.2888x over previous
import functools

import jax
import jax.numpy as jnp
from jax import lax
from jax.experimental import pallas as pl
from jax.experimental.pallas import tpu as pltpu
from jax.experimental.pallas import tpu_sc as plsc

N = 50000
E = 800000
D = 64
H = 32

NP = 50176
ROWS_PER_TILE = NP // 16
EP = 802816
CHUNKS = 784
CHUNKS_PER_TILE = CHUNKS // 16
DCHUNKS = 784
DCHUNKS_PER_TILE = DCHUNKS // 16
PAD_BASE = 50048

_mesh = plsc.VectorSubcoreMesh(core_axis_name="c", subcore_axis_name="s")
_sc_params = pltpu.CompilerParams(use_tc_tiling_on_sc=False)


def _zero_rows_vmem(rows_v, n_rows):
    z = jnp.zeros((16,), jnp.float32)

    def body(i, _):
        rows_v[i, pl.ds(0, 16)] = z
        rows_v[i, pl.ds(16, 16)] = z
        return 0

    lax.fori_loop(0, n_rows, body, 0)


@functools.partial(
    pl.kernel,
    out_type=(
        jax.ShapeDtypeStruct((NP,), jnp.float32),
        jax.ShapeDtypeStruct((NP,), jnp.float32),
    ),
    mesh=_mesh,
    scratch_types=(
        pltpu.VMEM((8, 128), jnp.int32),
        pltpu.VMEM((128,), jnp.float32),
        pltpu.VMEM((ROWS_PER_TILE,), jnp.float32),
        pltpu.VMEM_SHARED((NP,), jnp.float32),
    ),
    compiler_params=_sc_params,
)
def _degree_kernel(src_hbm, dst_hbm, dego_hbm, degi_hbm, idx_v, ones_v, buf_v, acc_sh):
    cid = lax.axis_index("c")
    sid = lax.axis_index("s")

    one = jnp.ones((16,), jnp.float32)
    zero = jnp.zeros((16,), jnp.float32)
    for j in range(8):
        ones_v[pl.ds(16 * j, 16)] = one

    def zbody(i, _):
        buf_v[pl.ds(16 * i, 16)] = zero
        return 0

    lax.fori_loop(0, ROWS_PER_TILE // 16, zbody, 0)

    base = sid * ROWS_PER_TILE
    pltpu.sync_copy(buf_v, acc_sh.at[pl.ds(base, ROWS_PER_TILE)])
    plsc.subcore_barrier()

    def accumulate(edge_hbm):
        def body(c, _):
            g = sid * DCHUNKS_PER_TILE + c
            pltpu.sync_copy(edge_hbm.at[g], idx_v)
            for j in range(8):
                pltpu.sync_copy(ones_v, acc_sh.at[idx_v.at[j]], add=True)
            return 0

        lax.fori_loop(0, DCHUNKS_PER_TILE, body, 0)

    @pl.when(cid == 0)
    def _():
        accumulate(src_hbm)

    @pl.when(cid == 1)
    def _():
        accumulate(dst_hbm)

    plsc.subcore_barrier()

    pltpu.sync_copy(acc_sh.at[pl.ds(base, ROWS_PER_TILE)], buf_v)

    @pl.when(cid == 0)
    def _():
        pltpu.sync_copy(buf_v, dego_hbm.at[pl.ds(base, ROWS_PER_TILE)])

    @pl.when(cid == 1)
    def _():
        pltpu.sync_copy(buf_v, degi_hbm.at[pl.ds(base, ROWS_PER_TILE)])


@functools.partial(
    pl.kernel,
    out_type=(
        jax.ShapeDtypeStruct((NP, H), jnp.float32),
        jax.ShapeDtypeStruct((NP, H), jnp.float32),
    ),
    mesh=_mesh,
    scratch_types=(
        pltpu.VMEM((8, 128), jnp.int32),
        pltpu.VMEM((8, 128), jnp.int32),
        pltpu.VMEM((8, 128), jnp.int32),
        pltpu.VMEM((8, 128), jnp.int32),
        pltpu.VMEM((128, H), jnp.float32),
        pltpu.VMEM((128, H), jnp.float32),
        pltpu.VMEM((128, H), jnp.float32),
        pltpu.VMEM((128, H), jnp.float32),
        pltpu.VMEM_SHARED((NP, H), jnp.float32),
        pltpu.SemaphoreType.DMA,
        pltpu.SemaphoreType.DMA,
        pltpu.SemaphoreType.DMA,
        pltpu.SemaphoreType.DMA,
        pltpu.SemaphoreType.DMA,
        pltpu.SemaphoreType.DMA,
        pltpu.SemaphoreType.DMA,
        pltpu.SemaphoreType.DMA,
        pltpu.SemaphoreType.DMA,
        pltpu.SemaphoreType.DMA,
    ),
    compiler_params=_sc_params,
)
def _message_kernel(h0_hbm, h1_hbm, src_hbm, dst_hbm, agg0_hbm, agg1_hbm,
                    src0, dst0, src1, dst1, r0, r1, r2, r3, acc_sh,
                    g0, g1, g2, g3, s0, s1, s2, s3, st0, st1):
    cid = lax.axis_index("c")
    sid = lax.axis_index("s")
    rows = [r0, r1, r2, r3]
    gs = [g0, g1, g2, g3]
    ss = [s0, s1, s2, s3]

    _zero_rows_vmem(r0, 128)
    base = sid * ROWS_PER_TILE
    for k in range(24):
        pltpu.sync_copy(r0, acc_sh.at[pl.ds(base + 128 * k, 128)])
    pltpu.sync_copy(r0.at[pl.ds(0, 64)],
                    acc_sh.at[pl.ds(base + 3072, 64)])
    plsc.subcore_barrier()

    def run(h_hbm):
        def gd(src_v, j):
            return pltpu.make_async_copy(h_hbm.at[src_v.at[j]],
                                         rows[j % 4], gs[j % 4])

        def sd(dst_v, j):
            return pltpu.make_async_copy(rows[j % 4],
                                         acc_sh.at[dst_v.at[j]], ss[j % 4])

        def stage_descs(c, src_v, dst_v, stsem):
            g = sid * CHUNKS_PER_TILE + c
            return (pltpu.make_async_copy(src_hbm.at[g], src_v, stsem),
                    pltpu.make_async_copy(dst_hbm.at[g], dst_v, stsem))

        def do_superchunk(k, sp, dp, sq, dq, stP, stQ):
            @pl.when(k > 0)
            def _():
                for j in range(4, 8):
                    sd(dq, j).wait()
            for d in stage_descs(lax.rem(k + 1, CHUNKS_PER_TILE), sq, dq, stQ):
                d.start()
            for d in stage_descs(k, sp, dp, stP):
                d.wait()
            for j in range(4):
                gd(sp, j).start()
            for j in range(8):
                gd(sp, j).wait()
                pltpu.async_copy(rows[j % 4], acc_sh.at[dp.at[j]], ss[j % 4],
                                 add=True)
                if j + 4 < 8:
                    sd(dp, j).wait()
                    gd(sp, j + 4).start()

        for d in stage_descs(0, src0, dst0, st0):
            d.start()

        def body(kk, _):
            do_superchunk(2 * kk, src0, dst0, src1, dst1, st0, st1)
            do_superchunk(2 * kk + 1, src1, dst1, src0, dst0, st1, st0)
            return 0

        lax.fori_loop(0, CHUNKS_PER_TILE // 2, body, 0)
        do_superchunk(jnp.int32(CHUNKS_PER_TILE - 1),
                      src0, dst0, src1, dst1, st0, st1)
        for j in range(4, 8):
            sd(dst0, j).wait()
        for d in stage_descs(0, src1, dst1, st1):
            d.wait()

    @pl.when(cid == 0)
    def _():
        run(h0_hbm)

    @pl.when(cid == 1)
    def _():
        run(h1_hbm)

    plsc.subcore_barrier()

    def writeout(agg_hbm):
        def wo(c, rv, sem):
            pltpu.sync_copy(acc_sh.at[pl.ds(base + 128 * c, 128)], rv)
            return pltpu.make_async_copy(rv, agg_hbm.at[pl.ds(base + 128 * c, 128)], sem)

        d_prev = None
        for k in range(24):
            d = wo(k, rows[k % 2], gs[k % 2])
            d.start()
            if d_prev is not None:
                d_prev.wait()
            d_prev = d
        d_prev.wait()
        pltpu.sync_copy(acc_sh.at[pl.ds(base + 3072, 64)],
                        r2.at[pl.ds(0, 64)])
        pltpu.sync_copy(r2.at[pl.ds(0, 64)],
                        agg_hbm.at[pl.ds(base + 3072, 64)])

    @pl.when(cid == 0)
    def _():
        writeout(agg0_hbm)

    @pl.when(cid == 1)
    def _():
        writeout(agg1_hbm)


_BLK = 2000
_GRID = N // _BLK


def _scale_body(x_ref, deg_ref, h0_ref, h1_ref):
    norm = lax.rsqrt(jnp.maximum(deg_ref[...], 1.0))
    h = x_ref[...] * norm
    h0_ref[...] = h[:, :H]
    h1_ref[...] = h[:, H:]


def _tc_scale(x, deg_out):
    return pl.pallas_call(
        _scale_body,
        grid=(_GRID,),
        in_specs=[
            pl.BlockSpec((_BLK, D), lambda i: (i, 0)),
            pl.BlockSpec((_BLK, 1), lambda i: (i, 0)),
        ],
        out_specs=[
            pl.BlockSpec((_BLK, H), lambda i: (i, 0)),
            pl.BlockSpec((_BLK, H), lambda i: (i, 0)),
        ],
        out_shape=[
            jax.ShapeDtypeStruct((N, H), jnp.float32),
            jax.ShapeDtypeStruct((N, H), jnp.float32),
        ],
    )(x, deg_out)


def _mid_body(a0_ref, a1_ref, din_ref, dout_ref, w_ref, b_ref, h0_ref, h1_ref):
    nd = lax.rsqrt(jnp.maximum(din_ref[...], 1.0))
    ns = lax.rsqrt(jnp.maximum(dout_ref[...], 1.0))
    a0 = a0_ref[...] * nd
    a1 = a1_ref[...] * nd
    h = (jnp.dot(a0, w_ref[:H, :], preferred_element_type=jnp.float32)
         + jnp.dot(a1, w_ref[H:, :], preferred_element_type=jnp.float32)
         + b_ref[...])
    h = jnp.maximum(h, 0.0) * ns
    h0_ref[...] = h[:, :H]
    h1_ref[...] = h[:, H:]


def _tc_mid(agg0, agg1, deg_in, deg_out, w, b):
    return pl.pallas_call(
        _mid_body,
        grid=(_GRID,),
        in_specs=[
            pl.BlockSpec((_BLK, H), lambda i: (i, 0)),
            pl.BlockSpec((_BLK, H), lambda i: (i, 0)),
            pl.BlockSpec((_BLK, 1), lambda i: (i, 0)),
            pl.BlockSpec((_BLK, 1), lambda i: (i, 0)),
            pl.BlockSpec((D, D), lambda i: (0, 0)),
            pl.BlockSpec((1, D), lambda i: (0, 0)),
        ],
        out_specs=[
            pl.BlockSpec((_BLK, H), lambda i: (i, 0)),
            pl.BlockSpec((_BLK, H), lambda i: (i, 0)),
        ],
        out_shape=[
            jax.ShapeDtypeStruct((N, H), jnp.float32),
            jax.ShapeDtypeStruct((N, H), jnp.float32),
        ],
    )(agg0, agg1, deg_in, deg_out, w, b)


def _final_body(a0_ref, a1_ref, din_ref, w_ref, b_ref, out_ref):
    nd = lax.rsqrt(jnp.maximum(din_ref[...], 1.0))
    a0 = a0_ref[...] * nd
    a1 = a1_ref[...] * nd
    out_ref[...] = (jnp.dot(a0, w_ref[:H, :], preferred_element_type=jnp.float32)
                    + jnp.dot(a1, w_ref[H:, :], preferred_element_type=jnp.float32)
                    + b_ref[...])


def _tc_final(agg0, agg1, deg_in, w, b):
    return pl.pallas_call(
        _final_body,
        grid=(_GRID,),
        in_specs=[
            pl.BlockSpec((_BLK, H), lambda i: (i, 0)),
            pl.BlockSpec((_BLK, H), lambda i: (i, 0)),
            pl.BlockSpec((_BLK, 1), lambda i: (i, 0)),
            pl.BlockSpec((D, D), lambda i: (0, 0)),
            pl.BlockSpec((1, D), lambda i: (0, 0)),
        ],
        out_specs=pl.BlockSpec((_BLK, D), lambda i: (i, 0)),
        out_shape=jax.ShapeDtypeStruct((N, D), jnp.float32),
    )(agg0, agg1, deg_in, w, b)


def kernel(inputs, edge_index, W1, b1, W2, b2):
    src = edge_index[0]
    dst = edge_index[1]
    padn = EP - E
    pad_ids = PAD_BASE + (jnp.arange(padn, dtype=jnp.int32) % 128)
    zeros_pad = jnp.zeros((padn,), jnp.int32)
    src_g = jnp.concatenate([src, zeros_pad]).reshape(CHUNKS, 8, 128)
    src_d = jnp.concatenate([src, pad_ids]).reshape(DCHUNKS, 8, 128)
    dst_s = jnp.concatenate([dst, pad_ids]).reshape(CHUNKS, 8, 128)
    dst_sd = dst_s

    deg_out, deg_in = _degree_kernel(src_d, dst_sd)
    deg_out = deg_out.reshape(NP, 1)
    deg_in = deg_in.reshape(NP, 1)

    b1r = b1.reshape(1, D)
    b2r = b2.reshape(1, D)

    h0, h1 = _tc_scale(inputs, deg_out)
    agg0, agg1 = _message_kernel(h0, h1, src_g, dst_s)
    h20, h21 = _tc_mid(agg0, agg1, deg_in, deg_out, W1, b1r)
    agg20, agg21 = _message_kernel(h20, h21, src_g, dst_s)
    return _tc_final(agg20, agg21, deg_in, W2, b2r)


# speedup vs baseline: 12.5221x; 1.1343x over previous
import functools

import jax
import jax.numpy as jnp
from jax import lax
from jax.experimental import pallas as pl
from jax.experimental.pallas import tpu as pltpu
from jax.experimental.pallas import tpu_sc as plsc

N = 50000
E = 800000
D = 64
H = 32

NP = 50176
ROWS_PER_TILE = NP // 16
EP = 802816
CHUNKS = 784
CHUNKS_PER_TILE = CHUNKS // 16
DCHUNKS = 784
DCHUNKS_PER_TILE = DCHUNKS // 16
PAD_BASE = 50048

_mesh = plsc.VectorSubcoreMesh(core_axis_name="c", subcore_axis_name="s")
_sc_params = pltpu.CompilerParams(use_tc_tiling_on_sc=False)


def _zero_rows_vmem(rows_v, n_rows):
    z = jnp.zeros((16,), jnp.float32)

    def body(i, _):
        rows_v[i, pl.ds(0, 16)] = z
        rows_v[i, pl.ds(16, 16)] = z
        return 0

    lax.fori_loop(0, n_rows, body, 0)


@functools.partial(
    pl.kernel,
    out_type=(
        jax.ShapeDtypeStruct((NP,), jnp.float32),
        jax.ShapeDtypeStruct((NP,), jnp.float32),
    ),
    mesh=_mesh,
    scratch_types=(
        pltpu.VMEM((8, 128), jnp.int32),
        pltpu.VMEM((8, 128), jnp.int32),
        pltpu.VMEM((128,), jnp.float32),
        pltpu.VMEM((ROWS_PER_TILE,), jnp.float32),
        pltpu.VMEM_SHARED((NP,), jnp.float32),
        pltpu.SemaphoreType.DMA,
        pltpu.SemaphoreType.DMA,
        pltpu.SemaphoreType.DMA,
        pltpu.SemaphoreType.DMA,
    ),
    compiler_params=_sc_params,
)
def _degree_kernel(src_hbm, dst_hbm, dego_hbm, degi_hbm, idxA, idxB, ones_v,
                   buf_v, acc_sh, stA, stB, ssA, ssB):
    cid = lax.axis_index("c")
    sid = lax.axis_index("s")

    one = jnp.ones((16,), jnp.float32)
    zero = jnp.zeros((16,), jnp.float32)
    for j in range(8):
        ones_v[pl.ds(16 * j, 16)] = one

    def zbody(i, _):
        buf_v[pl.ds(16 * i, 16)] = zero
        return 0

    lax.fori_loop(0, ROWS_PER_TILE // 16, zbody, 0)

    base = sid * ROWS_PER_TILE
    pltpu.sync_copy(buf_v, acc_sh.at[pl.ds(base, ROWS_PER_TILE)])
    plsc.subcore_barrier()

    def accumulate(edge_hbm):
        def stage_desc(c, idx_v, stsem):
            g = sid * DCHUNKS_PER_TILE + c
            return pltpu.make_async_copy(edge_hbm.at[g], idx_v, stsem)

        def sc_desc(idx_v, j, ssem):
            return pltpu.make_async_copy(ones_v, acc_sh.at[idx_v.at[j]], ssem)

        def do_chunk(k, iP, iQ, stP, stQ, sP, sQ):
            @pl.when(k > 0)
            def _():
                for j in range(8):
                    sc_desc(iQ, j, sQ).wait()
            stage_desc(lax.rem(k + 1, DCHUNKS_PER_TILE), iQ, stQ).start()
            stage_desc(k, iP, stP).wait()
            for j in range(8):
                pltpu.async_copy(ones_v, acc_sh.at[iP.at[j]], sP, add=True)

        stage_desc(0, idxA, stA).start()

        def body(kk, _):
            do_chunk(2 * kk, idxA, idxB, stA, stB, ssA, ssB)
            do_chunk(2 * kk + 1, idxB, idxA, stB, stA, ssB, ssA)
            return 0

        lax.fori_loop(0, DCHUNKS_PER_TILE // 2, body, 0)
        do_chunk(jnp.int32(DCHUNKS_PER_TILE - 1), idxA, idxB, stA, stB, ssA, ssB)
        for j in range(8):
            sc_desc(idxA, j, ssA).wait()
        stage_desc(0, idxB, stB).wait()

    @pl.when(cid == 0)
    def _():
        accumulate(src_hbm)

    @pl.when(cid == 1)
    def _():
        accumulate(dst_hbm)

    plsc.subcore_barrier()

    pltpu.sync_copy(acc_sh.at[pl.ds(base, ROWS_PER_TILE)], buf_v)

    @pl.when(cid == 0)
    def _():
        pltpu.sync_copy(buf_v, dego_hbm.at[pl.ds(base, ROWS_PER_TILE)])

    @pl.when(cid == 1)
    def _():
        pltpu.sync_copy(buf_v, degi_hbm.at[pl.ds(base, ROWS_PER_TILE)])


@functools.partial(
    pl.kernel,
    out_type=(
        jax.ShapeDtypeStruct((NP, H), jnp.float32),
        jax.ShapeDtypeStruct((NP, H), jnp.float32),
    ),
    mesh=_mesh,
    scratch_types=(
        pltpu.VMEM((8, 128), jnp.int32),
        pltpu.VMEM((8, 128), jnp.int32),
        pltpu.VMEM((8, 128), jnp.int32),
        pltpu.VMEM((8, 128), jnp.int32),
        pltpu.VMEM((128, H), jnp.float32),
        pltpu.VMEM((128, H), jnp.float32),
        pltpu.VMEM((128, H), jnp.float32),
        pltpu.VMEM((128, H), jnp.float32),
        pltpu.VMEM_SHARED((NP, H), jnp.float32),
        pltpu.SemaphoreType.DMA,
        pltpu.SemaphoreType.DMA,
        pltpu.SemaphoreType.DMA,
        pltpu.SemaphoreType.DMA,
        pltpu.SemaphoreType.DMA,
        pltpu.SemaphoreType.DMA,
        pltpu.SemaphoreType.DMA,
        pltpu.SemaphoreType.DMA,
        pltpu.SemaphoreType.DMA,
        pltpu.SemaphoreType.DMA,
    ),
    compiler_params=_sc_params,
)
def _message_kernel(h0_hbm, h1_hbm, src_hbm, dst_hbm, agg0_hbm, agg1_hbm,
                    src0, dst0, src1, dst1, r0, r1, r2, r3, acc_sh,
                    g0, g1, g2, g3, s0, s1, s2, s3, st0, st1):
    cid = lax.axis_index("c")
    sid = lax.axis_index("s")
    rows = [r0, r1, r2, r3]
    gs = [g0, g1, g2, g3]
    ss = [s0, s1, s2, s3]

    _zero_rows_vmem(r0, 128)
    base = sid * ROWS_PER_TILE
    for k in range(24):
        pltpu.sync_copy(r0, acc_sh.at[pl.ds(base + 128 * k, 128)])
    pltpu.sync_copy(r0.at[pl.ds(0, 64)],
                    acc_sh.at[pl.ds(base + 3072, 64)])
    plsc.subcore_barrier()

    def run(h_hbm):
        def gd(src_v, j):
            return pltpu.make_async_copy(h_hbm.at[src_v.at[j]],
                                         rows[j % 4], gs[j % 4])

        def sd(dst_v, j):
            return pltpu.make_async_copy(rows[j % 4],
                                         acc_sh.at[dst_v.at[j]], ss[j % 4])

        def stage_descs(c, src_v, dst_v, stsem):
            g = sid * CHUNKS_PER_TILE + c
            return (pltpu.make_async_copy(src_hbm.at[g], src_v, stsem),
                    pltpu.make_async_copy(dst_hbm.at[g], dst_v, stsem))

        def do_superchunk(k, sp, dp, sq, dq, stP, stQ):
            @pl.when(k > 0)
            def _():
                for j in range(4, 8):
                    sd(dq, j).wait()
            for d in stage_descs(lax.rem(k + 1, CHUNKS_PER_TILE), sq, dq, stQ):
                d.start()
            for d in stage_descs(k, sp, dp, stP):
                d.wait()
            for j in range(4):
                gd(sp, j).start()
            for j in range(8):
                gd(sp, j).wait()
                pltpu.async_copy(rows[j % 4], acc_sh.at[dp.at[j]], ss[j % 4],
                                 add=True)
                if j + 4 < 8:
                    sd(dp, j).wait()
                    gd(sp, j + 4).start()

        for d in stage_descs(0, src0, dst0, st0):
            d.start()

        def body(kk, _):
            do_superchunk(2 * kk, src0, dst0, src1, dst1, st0, st1)
            do_superchunk(2 * kk + 1, src1, dst1, src0, dst0, st1, st0)
            return 0

        lax.fori_loop(0, CHUNKS_PER_TILE // 2, body, 0)
        do_superchunk(jnp.int32(CHUNKS_PER_TILE - 1),
                      src0, dst0, src1, dst1, st0, st1)
        for j in range(4, 8):
            sd(dst0, j).wait()
        for d in stage_descs(0, src1, dst1, st1):
            d.wait()

    @pl.when(cid == 0)
    def _():
        run(h0_hbm)

    @pl.when(cid == 1)
    def _():
        run(h1_hbm)

    plsc.subcore_barrier()

    def writeout(agg_hbm):
        def wo(c, rv, sem):
            pltpu.sync_copy(acc_sh.at[pl.ds(base + 128 * c, 128)], rv)
            return pltpu.make_async_copy(rv, agg_hbm.at[pl.ds(base + 128 * c, 128)], sem)

        d_prev = None
        for k in range(24):
            d = wo(k, rows[k % 2], gs[k % 2])
            d.start()
            if d_prev is not None:
                d_prev.wait()
            d_prev = d
        d_prev.wait()
        pltpu.sync_copy(acc_sh.at[pl.ds(base + 3072, 64)],
                        r2.at[pl.ds(0, 64)])
        pltpu.sync_copy(r2.at[pl.ds(0, 64)],
                        agg_hbm.at[pl.ds(base + 3072, 64)])

    @pl.when(cid == 0)
    def _():
        writeout(agg0_hbm)

    @pl.when(cid == 1)
    def _():
        writeout(agg1_hbm)


_BLK = 2000
_GRID = N // _BLK


def _scale_body(x_ref, deg_ref, h0_ref, h1_ref):
    norm = lax.rsqrt(jnp.maximum(deg_ref[...], 1.0))
    h = x_ref[...] * norm
    h0_ref[...] = h[:, :H]
    h1_ref[...] = h[:, H:]


def _tc_scale(x, deg_out):
    return pl.pallas_call(
        _scale_body,
        grid=(_GRID,),
        in_specs=[
            pl.BlockSpec((_BLK, D), lambda i: (i, 0)),
            pl.BlockSpec((_BLK, 1), lambda i: (i, 0)),
        ],
        out_specs=[
            pl.BlockSpec((_BLK, H), lambda i: (i, 0)),
            pl.BlockSpec((_BLK, H), lambda i: (i, 0)),
        ],
        out_shape=[
            jax.ShapeDtypeStruct((NP, H), jnp.float32),
            jax.ShapeDtypeStruct((NP, H), jnp.float32),
        ],
    )(x, deg_out)


def _mid_body(a0_ref, a1_ref, din_ref, dout_ref, w_ref, b_ref, h0_ref, h1_ref):
    nd = lax.rsqrt(jnp.maximum(din_ref[...], 1.0))
    ns = lax.rsqrt(jnp.maximum(dout_ref[...], 1.0))
    a0 = a0_ref[...] * nd
    a1 = a1_ref[...] * nd
    h = (jnp.dot(a0, w_ref[:H, :], preferred_element_type=jnp.float32)
         + jnp.dot(a1, w_ref[H:, :], preferred_element_type=jnp.float32)
         + b_ref[...])
    h = jnp.maximum(h, 0.0) * ns
    h0_ref[...] = h[:, :H]
    h1_ref[...] = h[:, H:]


def _tc_mid(agg0, agg1, deg_in_col, deg_out_col, w, b):
    return pl.pallas_call(
        _mid_body,
        grid=(_GRID,),
        in_specs=[
            pl.BlockSpec((_BLK, H), lambda i: (i, 0)),
            pl.BlockSpec((_BLK, H), lambda i: (i, 0)),
            pl.BlockSpec((_BLK, 1), lambda i: (i, 0)),
            pl.BlockSpec((_BLK, 1), lambda i: (i, 0)),
            pl.BlockSpec((D, D), lambda i: (0, 0)),
            pl.BlockSpec((1, D), lambda i: (0, 0)),
        ],
        out_specs=[
            pl.BlockSpec((_BLK, H), lambda i: (i, 0)),
            pl.BlockSpec((_BLK, H), lambda i: (i, 0)),
        ],
        out_shape=[
            jax.ShapeDtypeStruct((NP, H), jnp.float32),
            jax.ShapeDtypeStruct((NP, H), jnp.float32),
        ],
    )(agg0, agg1, deg_in_col, deg_out_col, w, b)


def _final_body(a0_ref, a1_ref, din_ref, w_ref, b_ref, out_ref):
    nd = lax.rsqrt(jnp.maximum(din_ref[...], 1.0))
    a0 = a0_ref[...] * nd
    a1 = a1_ref[...] * nd
    out_ref[...] = (jnp.dot(a0, w_ref[:H, :], preferred_element_type=jnp.float32)
                    + jnp.dot(a1, w_ref[H:, :], preferred_element_type=jnp.float32)
                    + b_ref[...])


def _tc_final(agg0, agg1, deg_in, w, b):
    return pl.pallas_call(
        _final_body,
        grid=(_GRID,),
        in_specs=[
            pl.BlockSpec((_BLK, H), lambda i: (i, 0)),
            pl.BlockSpec((_BLK, H), lambda i: (i, 0)),
            pl.BlockSpec((_BLK, 1), lambda i: (i, 0)),
            pl.BlockSpec((D, D), lambda i: (0, 0)),
            pl.BlockSpec((1, D), lambda i: (0, 0)),
        ],
        out_specs=pl.BlockSpec((_BLK, D), lambda i: (i, 0)),
        out_shape=jax.ShapeDtypeStruct((N, D), jnp.float32),
    )(agg0, agg1, deg_in, w, b)


def kernel(inputs, edge_index, W1, b1, W2, b2):
    src = edge_index[0]
    dst = edge_index[1]
    padn = EP - E
    pad_ids = PAD_BASE + (jnp.arange(padn, dtype=jnp.int32) % 128)
    src_p = jnp.concatenate([src, pad_ids]).reshape(CHUNKS, 8, 128)
    dst_p = jnp.concatenate([dst, pad_ids]).reshape(CHUNKS, 8, 128)

    deg_out, deg_in = _degree_kernel(src_p, dst_p)
    deg_out_col = deg_out.reshape(NP, 1)
    deg_in_col = deg_in.reshape(NP, 1)

    b1r = b1.reshape(1, D)
    b2r = b2.reshape(1, D)

    h0, h1 = _tc_scale(inputs, deg_out_col)
    agg0, agg1 = _message_kernel(h0, h1, src_p, dst_p)
    h20, h21 = _tc_mid(agg0, agg1, deg_in_col, deg_out_col, W1, b1r)
    agg20, agg21 = _message_kernel(h20, h21, src_p, dst_p)
    return _tc_final(agg20, agg21, deg_in_col, W2, b2r)
